# baseline (speedup 1.0000x reference)
import functools

import jax
import jax.numpy as jnp
from jax import lax
from jax.experimental import pallas as pl
from jax.experimental.pallas import tpu as pltpu
from jax.experimental.pallas import tpu_sc as plsc

EMB = 128
TOKENS = 20
LANES = 16
CHUNKS = EMB // LANES

NODES_PER_GATHER = 4
IDX_PER_GATHER = NODES_PER_GATHER * TOKENS
GATHERS_PER_STEP = 4
NODES_PER_STEP = NODES_PER_GATHER * GATHERS_PER_STEP
ROWS_PER_STEP = NODES_PER_STEP * TOKENS
NBUF = 2


def _build_sc_kernel(batch, n_workers):
    nodes_per_tile = batch // n_workers
    steps = nodes_per_tile // NODES_PER_STEP
    idx_rows_per_tile = nodes_per_tile // NODES_PER_GATHER

    mesh = plsc.VectorSubcoreMesh(core_axis_name="c", subcore_axis_name="s")
    nc = 2

    @functools.partial(
        pl.kernel,
        mesh=mesh,
        out_type=jax.ShapeDtypeStruct((batch, EMB), jnp.float32),
        scratch_types=[
            pltpu.VMEM((idx_rows_per_tile, IDX_PER_GATHER), jnp.int32),
        ]
        + [pltpu.VMEM((ROWS_PER_STEP, EMB), jnp.float32) for _ in range(NBUF)]
        + [pltpu.VMEM((NODES_PER_STEP, EMB), jnp.float32) for _ in range(NBUF)]
        + [pltpu.SemaphoreType.DMA for _ in range(NBUF)]
        + [pltpu.SemaphoreType.DMA for _ in range(NBUF)],
    )
    def emb_kernel(tok_hbm, table_hbm, out_hbm, idx_v, *bufs_and_sems):
        rows_bufs = bufs_and_sems[:NBUF]
        out_bufs = bufs_and_sems[NBUF:2 * NBUF]
        sems = bufs_and_sems[2 * NBUF:3 * NBUF]
        out_sems = bufs_and_sems[3 * NBUF:]
        i32 = lambda v: jnp.int32(v)
        wid = lax.axis_index("s") * i32(nc) + lax.axis_index("c")
        idx_row0 = wid * i32(idx_rows_per_tile)
        node0 = wid * i32(nodes_per_tile)

        pltpu.sync_copy(tok_hbm.at[pl.ds(idx_row0, idx_rows_per_tile)], idx_v)

        def fire(s, rows_v, sem):
            for j in range(GATHERS_PER_STEP):
                pltpu.async_copy(
                    table_hbm.at[idx_v.at[s * i32(GATHERS_PER_STEP) + i32(j)]],
                    rows_v.at[pl.ds(j * IDX_PER_GATHER, IDX_PER_GATHER)],
                    sem,
                )

        def drain(s, rows_v, sem):
            for j in range(GATHERS_PER_STEP):
                pltpu.make_async_copy(
                    table_hbm.at[idx_v.at[s * i32(GATHERS_PER_STEP) + i32(j)]],
                    rows_v.at[pl.ds(j * IDX_PER_GATHER, IDX_PER_GATHER)],
                    sem,
                ).wait()

        def out_slice(s):
            return out_hbm.at[
                pl.ds(node0 + s * i32(NODES_PER_STEP), NODES_PER_STEP)
            ]

        def compute(s, rows_v, out_v, out_sem):
            @pl.when(s >= i32(NBUF))
            def _():
                pltpu.make_async_copy(
                    out_v, out_slice(s - i32(NBUF)), out_sem
                ).wait()

            def node4(q, carry2):
                base4 = q * i32(4 * TOKENS)
                for g in range(4):
                    for c in range(CHUNKS):
                        sl = pl.ds(c * LANES, LANES)
                        acc = rows_v[base4 + i32(g * TOKENS), sl]
                        for t in range(1, TOKENS):
                            acc = acc + rows_v[base4 + i32(g * TOKENS + t), sl]
                        out_v[q * i32(4) + i32(g), sl] = acc
                return carry2

            lax.fori_loop(0, jnp.int32(NODES_PER_STEP // 4), node4, 0,
                          unroll=False)
            pltpu.async_copy(out_v, out_slice(s), out_sem)

        for b in range(NBUF):
            fire(i32(b), rows_bufs[b], sems[b])

        def ring(it, carry):
            for b in range(NBUF):
                s = it * i32(NBUF) + i32(b)
                drain(s, rows_bufs[b], sems[b])
                compute(s, rows_bufs[b], out_bufs[b], out_sems[b])

                @pl.when(s + i32(NBUF) < i32(steps))
                def _():
                    fire(s + i32(NBUF), rows_bufs[b], sems[b])

            return carry

        lax.fori_loop(0, jnp.int32(steps // NBUF), ring, 0, unroll=False)

        for b in range(NBUF):
            s_last = i32(steps - NBUF + b)
            pltpu.make_async_copy(
                out_bufs[b], out_slice(s_last), out_sems[b]
            ).wait()

    return emb_kernel


def kernel(buckets, node_ids, token_ids):
    del node_ids
    batch = token_ids.shape[0]
    n_workers = 32
    tok = token_ids.astype(jnp.int32).reshape(
        batch * TOKENS // IDX_PER_GATHER, IDX_PER_GATHER
    )
    emb_kernel = _build_sc_kernel(batch, n_workers)
    return emb_kernel(tok, buckets)


# speedup vs baseline: 2.3521x; 2.3521x over previous
import functools

import jax
import jax.numpy as jnp
from jax import lax
from jax.experimental import pallas as pl
from jax.experimental.pallas import tpu as pltpu
from jax.experimental.pallas import tpu_sc as plsc

EMB = 128
TOKENS = 20
LANES = 16
CHUNKS = EMB // LANES

NODES_PER_GATHER = 4
IDX_PER_GATHER = NODES_PER_GATHER * TOKENS
GATHERS_PER_STEP = 4
NODES_PER_STEP = NODES_PER_GATHER * GATHERS_PER_STEP
ROWS_PER_STEP = NODES_PER_STEP * TOKENS
NBUF = 2


def _build_sc_kernel(batch, n_workers):
    nodes_per_tile = batch // n_workers
    steps = nodes_per_tile // NODES_PER_STEP
    idx_rows_per_tile = nodes_per_tile // NODES_PER_GATHER

    mesh = plsc.VectorSubcoreMesh(core_axis_name="c", subcore_axis_name="s")
    nc = 2

    @functools.partial(
        pl.kernel,
        mesh=mesh,
        out_type=jax.ShapeDtypeStruct((batch, EMB), jnp.float32),
        scratch_types=[
            pltpu.VMEM((idx_rows_per_tile, IDX_PER_GATHER), jnp.int32),
        ]
        + [pltpu.VMEM((ROWS_PER_STEP, EMB), jnp.float32) for _ in range(NBUF)]
        + [pltpu.VMEM((NODES_PER_STEP, EMB), jnp.float32) for _ in range(NBUF)]
        + [pltpu.SemaphoreType.DMA for _ in range(NBUF)]
        + [pltpu.SemaphoreType.DMA for _ in range(NBUF)],
    )
    def emb_kernel(tok_hbm, table_hbm, out_hbm, idx_v, *bufs_and_sems):
        rows_bufs = bufs_and_sems[:NBUF]
        out_bufs = bufs_and_sems[NBUF:2 * NBUF]
        sems = bufs_and_sems[2 * NBUF:3 * NBUF]
        out_sems = bufs_and_sems[3 * NBUF:]
        i32 = lambda v: jnp.int32(v)
        wid = lax.axis_index("s") * i32(nc) + lax.axis_index("c")
        idx_row0 = wid * i32(idx_rows_per_tile)
        node0 = wid * i32(nodes_per_tile)

        pltpu.sync_copy(tok_hbm.at[pl.ds(idx_row0, idx_rows_per_tile)], idx_v)

        def fire(s, rows_v, sem):
            for j in range(GATHERS_PER_STEP):
                pltpu.async_copy(
                    table_hbm.at[idx_v.at[s * i32(GATHERS_PER_STEP) + i32(j)]],
                    rows_v.at[pl.ds(j * IDX_PER_GATHER, IDX_PER_GATHER)],
                    sem,
                )

        def drain(s, rows_v, sem):
            for j in range(GATHERS_PER_STEP):
                pltpu.make_async_copy(
                    table_hbm.at[idx_v.at[s * i32(GATHERS_PER_STEP) + i32(j)]],
                    rows_v.at[pl.ds(j * IDX_PER_GATHER, IDX_PER_GATHER)],
                    sem,
                ).wait()

        def out_slice(s):
            return out_hbm.at[
                pl.ds(node0 + s * i32(NODES_PER_STEP), NODES_PER_STEP)
            ]

        def compute(s, rows_v, out_v, out_sem):
            @pl.when(s >= i32(NBUF))
            def _():
                pltpu.make_async_copy(
                    out_v, out_slice(s - i32(NBUF)), out_sem
                ).wait()

            def node(g, carry2):
                base = g * i32(TOKENS)
                sls = [pl.ds(c * LANES, LANES) for c in range(CHUNKS)]
                accs = [rows_v[base, sl] for sl in sls]
                for t in range(1, TOKENS):
                    row = base + i32(t)
                    accs = [acc + rows_v[row, sl]
                            for acc, sl in zip(accs, sls)]
                for c, sl in enumerate(sls):
                    out_v[g, sl] = accs[c]
                return carry2

            lax.fori_loop(0, jnp.int32(NODES_PER_STEP), node, 0, unroll=False)
            pltpu.async_copy(out_v, out_slice(s), out_sem)

        for b in range(NBUF):
            fire(i32(b), rows_bufs[b], sems[b])

        def ring(it, carry):
            for b in range(NBUF):
                s = it * i32(NBUF) + i32(b)
                drain(s, rows_bufs[b], sems[b])
                compute(s, rows_bufs[b], out_bufs[b], out_sems[b])

                @pl.when(s + i32(NBUF) < i32(steps))
                def _():
                    fire(s + i32(NBUF), rows_bufs[b], sems[b])

            return carry

        lax.fori_loop(0, jnp.int32(steps // NBUF), ring, 0, unroll=False)

        for b in range(NBUF):
            s_last = i32(steps - NBUF + b)
            pltpu.make_async_copy(
                out_bufs[b], out_slice(s_last), out_sems[b]
            ).wait()

    return emb_kernel


def kernel(buckets, node_ids, token_ids):
    del node_ids
    batch = token_ids.shape[0]
    n_workers = 32
    tok = token_ids.astype(jnp.int32).reshape(
        batch * TOKENS // IDX_PER_GATHER, IDX_PER_GATHER
    )
    emb_kernel = _build_sc_kernel(batch, n_workers)
    return emb_kernel(tok, buckets)
